# baseline (speedup 1.0000x reference)
import functools

import jax
import jax.numpy as jnp
from jax.experimental import pallas as pl

_BB = 32


def _mix(adj3, s3, nj):
    acc = adj3[:, :, 0:1] * s3[:, 0:1, :]
    for j in range(1, nj):
        acc = acc + adj3[:, :, j : j + 1] * s3[:, j : j + 1, :]
    return acc


def _gcn_kernel(x_ref, adj_ref, pose_ref, padj_ref,
                w1_ref, b1_ref, w3_ref, b3_ref,
                wp1_ref, bp1_ref, wp3_ref, bp3_ref,
                fcw_ref, fcb_ref, out_ref):
    bb = adj_ref.shape[0]

    xb = x_ref[...]
    adj = adj_ref[...]
    s1 = jnp.dot(xb, w1_ref[...], preferred_element_type=jnp.float32)
    s1 = s1.reshape(bb, 8, s1.shape[-1])
    h1 = jnp.maximum(_mix(adj, s1, 8) + b1_ref[...].reshape(1, 1, -1), 0.0)
    s2 = jnp.dot(h1.reshape(bb * 8, h1.shape[-1]), w3_ref[...],
                 preferred_element_type=jnp.float32)
    s2 = s2.reshape(bb, 8, s2.shape[-1])
    h2 = jnp.maximum(_mix(adj, s2, 8) + b3_ref[...].reshape(1, 1, -1), 0.0)
    hmean = jnp.mean(h2, axis=1)

    pb = pose_ref[...]
    padj = padj_ref[...]
    sp1 = jnp.dot(pb, wp1_ref[...], preferred_element_type=jnp.float32)
    sp1 = sp1.reshape(bb, 40, sp1.shape[-1])
    p1 = jnp.maximum(_mix(padj, sp1, 34) + bp1_ref[...].reshape(1, 1, -1), 0.0)
    sp2 = jnp.dot(p1.reshape(bb * 40, p1.shape[-1]), wp3_ref[...],
                  preferred_element_type=jnp.float32)
    sp2 = sp2.reshape(bb, 40, sp2.shape[-1])
    p2 = jnp.maximum(_mix(padj, sp2, 34) + bp3_ref[...].reshape(1, 1, -1), 0.0)
    pmean = jnp.sum(p2[:, :34, :], axis=1) * (1.0 / 34.0)

    feat = jnp.concatenate([hmean, pmean], axis=1)
    out_ref[...] = (jnp.dot(feat, fcw_ref[...], preferred_element_type=jnp.float32)
                    + fcb_ref[...])


@functools.partial(jax.jit, static_argnames=())
def kernel(x, adj, pose, pose_adj, W1, b1, W3, b3, Wp1, bp1, Wp3, bp3, fcW, fcb):
    B = x.shape[0]
    bb = _BB
    nblk = B // bb

    xf = x.reshape(B * 8, x.shape[-1])
    pose_p = jnp.pad(pose, ((0, 0), (0, 6), (0, 0)))
    posef = pose_p.reshape(B * 40, pose.shape[-1])
    padj = jnp.pad(pose_adj, ((0, 0), (0, 6), (0, 6)))

    b1r = b1.reshape(1, -1)
    b3r = b3.reshape(1, -1)
    bp1r = bp1.reshape(1, -1)
    bp3r = bp3.reshape(1, -1)
    fcbr = fcb.reshape(1, -1)

    const2 = lambda i: (0, 0)

    out = pl.pallas_call(
        _gcn_kernel,
        grid=(nblk,),
        in_specs=[
            pl.BlockSpec((bb * 8, x.shape[-1]), lambda i: (i, 0)),
            pl.BlockSpec((bb, 8, 8), lambda i: (i, 0, 0)),
            pl.BlockSpec((bb * 40, pose.shape[-1]), lambda i: (i, 0)),
            pl.BlockSpec((bb, 40, 40), lambda i: (i, 0, 0)),
            pl.BlockSpec(W1.shape, const2),
            pl.BlockSpec(b1r.shape, const2),
            pl.BlockSpec(W3.shape, const2),
            pl.BlockSpec(b3r.shape, const2),
            pl.BlockSpec(Wp1.shape, const2),
            pl.BlockSpec(bp1r.shape, const2),
            pl.BlockSpec(Wp3.shape, const2),
            pl.BlockSpec(bp3r.shape, const2),
            pl.BlockSpec(fcW.shape, const2),
            pl.BlockSpec(fcbr.shape, const2),
        ],
        out_specs=pl.BlockSpec((bb, fcW.shape[-1]), lambda i: (i, 0)),
        out_shape=jax.ShapeDtypeStruct((B, fcW.shape[-1]), jnp.float32),
    )(xf, adj, posef, padj, W1, b1r, W3, b3r, Wp1, bp1r, Wp3, bp3r, fcW, fcbr)
    return out


# speedup vs baseline: 2.1908x; 2.1908x over previous
import functools

import jax
import jax.numpy as jnp
import numpy as np
from jax.experimental import pallas as pl

_BB = 32
_GX = 16
_GP = 4


def _gcn_kernel(x_ref, ax_ref, pose_ref, ap_ref,
                w1_ref, b1_ref, w3_ref, b3_ref,
                wp1_ref, bp1_ref, wp3_ref, bp3_ref,
                fcw_ref, fcb_ref, msel_ref, out_ref):
    f32 = jnp.float32
    bf16 = jnp.bfloat16
    ngx = _BB // _GX
    rgx = _GX * 8
    ngp = _BB // _GP
    rgp = _GP * 34

    def mix(a_ref, s, ng, rg):
        sb = s.astype(bf16)
        parts = [jnp.dot(a_ref[k], sb[k * rg:(k + 1) * rg],
                         preferred_element_type=f32)
                 for k in range(ng)]
        return jnp.concatenate(parts, axis=0)

    xb = x_ref[...].astype(bf16)
    s1 = jnp.dot(xb, w1_ref[...], preferred_element_type=f32)
    h1 = jnp.maximum(mix(ax_ref, s1, ngx, rgx) + b1_ref[...], 0.0)
    s2 = jnp.dot(h1.astype(bf16), w3_ref[...], preferred_element_type=f32)
    h2 = jnp.maximum(mix(ax_ref, s2, ngx, rgx) + b3_ref[...], 0.0)
    hmean = jnp.mean(h2.reshape(_BB, 8, h2.shape[-1]), axis=1)

    pb = pose_ref[...]
    pm = mix(ap_ref, pb, ngp, rgp)
    sp1 = jnp.dot(pm.astype(bf16), wp1_ref[...], preferred_element_type=f32)
    p1 = jnp.maximum(sp1 + bp1_ref[...], 0.0)
    sp2 = jnp.dot(p1.astype(bf16), wp3_ref[...], preferred_element_type=f32)
    p2 = jnp.maximum(mix(ap_ref, sp2, ngp, rgp) + bp3_ref[...], 0.0)
    pmean = jnp.dot(msel_ref[...], p2.astype(bf16), preferred_element_type=f32)

    feat = jnp.concatenate([hmean, pmean], axis=1)
    out_ref[...] = (jnp.dot(feat.astype(bf16), fcw_ref[...],
                            preferred_element_type=f32) + fcb_ref[...])


def _blockdiag(a, g):
    B, n, _ = a.shape
    ar = a.reshape(B // g, g, n, n)
    eye = jnp.eye(g, dtype=jnp.bool_)
    big = jnp.where(eye[None, :, None, :, None], ar[:, :, :, None, :], 0.0)
    return big.reshape(B // g, g * n, g * n).astype(jnp.bfloat16)


@jax.jit
def kernel(x, adj, pose, pose_adj, W1, b1, W3, b3, Wp1, bp1, Wp3, bp3, fcW, fcb):
    B = x.shape[0]
    bb = _BB
    nblk = B // bb

    xf = x.reshape(B * 8, x.shape[-1])
    posef = pose.reshape(B * 34, pose.shape[-1])
    ax = _blockdiag(adj, _GX)
    ap = _blockdiag(pose_adj, _GP)

    bf16 = jnp.bfloat16
    w1b, w3b = W1.astype(bf16), W3.astype(bf16)
    wp1b, wp3b = Wp1.astype(bf16), Wp3.astype(bf16)
    fcwb = fcW.astype(bf16)
    b1r, b3r = b1.reshape(1, -1), b3.reshape(1, -1)
    bp1r, bp3r = bp1.reshape(1, -1), bp3.reshape(1, -1)
    fcbr = fcb.reshape(1, -1)

    msel_np = np.zeros((bb, bb * 34), np.float32)
    for i in range(bb):
        msel_np[i, i * 34:(i + 1) * 34] = 1.0 / 34.0
    msel = jnp.asarray(msel_np, dtype=bf16)

    const2 = lambda i: (0, 0)
    const3 = lambda i: (0, 0, 0)

    out = pl.pallas_call(
        _gcn_kernel,
        grid=(nblk,),
        in_specs=[
            pl.BlockSpec((bb * 8, x.shape[-1]), lambda i: (i, 0)),
            pl.BlockSpec((bb // _GX, _GX * 8, _GX * 8), lambda i: (i, 0, 0)),
            pl.BlockSpec((bb * 34, pose.shape[-1]), lambda i: (i, 0)),
            pl.BlockSpec((bb // _GP, _GP * 34, _GP * 34), lambda i: (i, 0, 0)),
            pl.BlockSpec(w1b.shape, const2),
            pl.BlockSpec(b1r.shape, const2),
            pl.BlockSpec(w3b.shape, const2),
            pl.BlockSpec(b3r.shape, const2),
            pl.BlockSpec(wp1b.shape, const2),
            pl.BlockSpec(bp1r.shape, const2),
            pl.BlockSpec(wp3b.shape, const2),
            pl.BlockSpec(bp3r.shape, const2),
            pl.BlockSpec(fcwb.shape, const2),
            pl.BlockSpec(fcbr.shape, const2),
            pl.BlockSpec(msel.shape, const2),
        ],
        out_specs=pl.BlockSpec((bb, fcW.shape[-1]), lambda i: (i, 0)),
        out_shape=jax.ShapeDtypeStruct((B, fcW.shape[-1]), jnp.float32),
    )(xf, ax, posef, ap, w1b, b1r, w3b, b3r, wp1b, bp1r, wp3b, bp3r,
      fcwb, fcbr, msel)
    return out


# speedup vs baseline: 3.0031x; 1.3708x over previous
import functools

import jax
import jax.numpy as jnp
import numpy as np
from jax.experimental import pallas as pl

_BB = 32
_GX = 16
_GP = 4


def _blkdiag_bf16(ablk, g, n):
    total = g * n
    acc = None
    for m in range(g):
        pw = ((m * n, total - (m + 1) * n), (m * n, total - (m + 1) * n))
        piece = jnp.pad(ablk[m], pw)
        acc = piece if acc is None else acc + piece
    return acc.astype(jnp.bfloat16)


def _gcn_kernel(x_ref, adj_ref, pose_ref, padj_ref,
                w1_ref, b1_ref, w3_ref, b3_ref,
                wp1_ref, bp1_ref, wp3_ref, bp3_ref,
                fcw_ref, fcb_ref, msel_ref, out_ref):
    f32 = jnp.float32
    bf16 = jnp.bfloat16
    ngx = _BB // _GX
    rgx = _GX * 8
    ngp = _BB // _GP
    rgp = _GP * 34

    def mix(amats, s, rg):
        sb = s.astype(bf16)
        parts = [jnp.dot(a, sb[k * rg:(k + 1) * rg],
                         preferred_element_type=f32)
                 for k, a in enumerate(amats)]
        return jnp.concatenate(parts, axis=0)

    adjb = adj_ref[...]
    ax = [_blkdiag_bf16(adjb[k * _GX:(k + 1) * _GX], _GX, 8)
          for k in range(ngx)]
    padjb = padj_ref[...]
    ap = [_blkdiag_bf16(padjb[k * _GP:(k + 1) * _GP], _GP, 34)
          for k in range(ngp)]

    xb = x_ref[...].astype(bf16)
    s1 = jnp.dot(xb, w1_ref[...], preferred_element_type=f32)
    h1 = jnp.maximum(mix(ax, s1, rgx) + b1_ref[...], 0.0)
    s2 = jnp.dot(h1.astype(bf16), w3_ref[...], preferred_element_type=f32)
    h2 = jnp.maximum(mix(ax, s2, rgx) + b3_ref[...], 0.0)
    hmean = jnp.mean(h2.reshape(_BB, 8, h2.shape[-1]), axis=1)

    pb = pose_ref[...]
    pm = mix(ap, pb, rgp)
    sp1 = jnp.dot(pm.astype(bf16), wp1_ref[...], preferred_element_type=f32)
    p1 = jnp.maximum(sp1 + bp1_ref[...], 0.0)
    sp2 = jnp.dot(p1.astype(bf16), wp3_ref[...], preferred_element_type=f32)
    p2 = jnp.maximum(mix(ap, sp2, rgp) + bp3_ref[...], 0.0)
    pmean = jnp.dot(msel_ref[...], p2.astype(bf16), preferred_element_type=f32)

    feat = jnp.concatenate([hmean, pmean], axis=1)
    out_ref[...] = (jnp.dot(feat.astype(bf16), fcw_ref[...],
                            preferred_element_type=f32) + fcb_ref[...])


@jax.jit
def kernel(x, adj, pose, pose_adj, W1, b1, W3, b3, Wp1, bp1, Wp3, bp3, fcW, fcb):
    B = x.shape[0]
    bb = _BB
    nblk = B // bb

    xf = x.reshape(B * 8, x.shape[-1])
    posef = pose.reshape(B * 34, pose.shape[-1])

    bf16 = jnp.bfloat16
    w1b, w3b = W1.astype(bf16), W3.astype(bf16)
    wp1b, wp3b = Wp1.astype(bf16), Wp3.astype(bf16)
    fcwb = fcW.astype(bf16)
    b1r, b3r = b1.reshape(1, -1), b3.reshape(1, -1)
    bp1r, bp3r = bp1.reshape(1, -1), bp3.reshape(1, -1)
    fcbr = fcb.reshape(1, -1)

    msel_np = np.zeros((bb, bb * 34), np.float32)
    for i in range(bb):
        msel_np[i, i * 34:(i + 1) * 34] = 1.0 / 34.0
    msel = jnp.asarray(msel_np, dtype=bf16)

    const2 = lambda i: (0, 0)
    const3 = lambda i: (0, 0, 0)

    out = pl.pallas_call(
        _gcn_kernel,
        grid=(nblk,),
        in_specs=[
            pl.BlockSpec((bb * 8, x.shape[-1]), lambda i: (i, 0)),
            pl.BlockSpec((bb, 8, 8), lambda i: (i, 0, 0)),
            pl.BlockSpec((bb * 34, pose.shape[-1]), lambda i: (i, 0)),
            pl.BlockSpec((bb, 34, 34), lambda i: (i, 0, 0)),
            pl.BlockSpec(w1b.shape, const2),
            pl.BlockSpec(b1r.shape, const2),
            pl.BlockSpec(w3b.shape, const2),
            pl.BlockSpec(b3r.shape, const2),
            pl.BlockSpec(wp1b.shape, const2),
            pl.BlockSpec(bp1r.shape, const2),
            pl.BlockSpec(wp3b.shape, const2),
            pl.BlockSpec(bp3r.shape, const2),
            pl.BlockSpec(fcwb.shape, const2),
            pl.BlockSpec(fcbr.shape, const2),
            pl.BlockSpec(msel.shape, const2),
        ],
        out_specs=pl.BlockSpec((bb, fcW.shape[-1]), lambda i: (i, 0)),
        out_shape=jax.ShapeDtypeStruct((B, fcW.shape[-1]), jnp.float32),
    )(xf, adj, posef, pose_adj, w1b, b1r, w3b, b3r, wp1b, bp1r, wp3b, bp3r,
      fcwb, fcbr, msel)
    return out


# speedup vs baseline: 3.0815x; 1.0261x over previous
import functools

import jax
import jax.numpy as jnp
import numpy as np
from jax.experimental import pallas as pl
from jax.experimental.pallas import tpu as pltpu

_BB = 32
_GX = 16
_GP = 4


def _blkdiag_bf16(ablk, g, n):
    total = g * n
    acc = None
    for m in range(g):
        pw = ((m * n, total - (m + 1) * n), (m * n, total - (m + 1) * n))
        piece = jnp.pad(ablk[m], pw)
        acc = piece if acc is None else acc + piece
    return acc.astype(jnp.bfloat16)


def _gcn_kernel(x_ref, adj_ref, pose_ref, padj_ref,
                w1_ref, b1_ref, w3_ref, b3_ref,
                wp1_ref, bp1_ref, wp3_ref, bp3_ref,
                fcw_ref, fcb_ref, msel_ref, out_ref,
                w1s_ref, w3s_ref, wp1s_ref, wp3s_ref, fcws_ref):
    f32 = jnp.float32
    bf16 = jnp.bfloat16

    @pl.when(pl.program_id(0) == 0)
    def _cast_weights():
        w1s_ref[...] = w1_ref[...].astype(bf16)
        w3s_ref[...] = w3_ref[...].astype(bf16)
        wp1s_ref[...] = wp1_ref[...].astype(bf16)
        wp3s_ref[...] = wp3_ref[...].astype(bf16)
        fcws_ref[...] = fcw_ref[...].astype(bf16)
    ngx = _BB // _GX
    rgx = _GX * 8
    ngp = _BB // _GP
    rgp = _GP * 34

    def mix(amats, s, rg):
        sb = s.astype(bf16)
        parts = [jnp.dot(a, sb[k * rg:(k + 1) * rg],
                         preferred_element_type=f32)
                 for k, a in enumerate(amats)]
        return jnp.concatenate(parts, axis=0)

    adjb = adj_ref[...]
    ax = [_blkdiag_bf16(adjb[k * _GX:(k + 1) * _GX], _GX, 8)
          for k in range(ngx)]
    padjb = padj_ref[...]
    ap = [_blkdiag_bf16(padjb[k * _GP:(k + 1) * _GP], _GP, 34)
          for k in range(ngp)]

    xb = x_ref[...].astype(bf16)
    s1 = jnp.dot(xb, w1s_ref[...], preferred_element_type=f32)
    h1 = jnp.maximum(mix(ax, s1, rgx) + b1_ref[...], 0.0)
    s2 = jnp.dot(h1.astype(bf16), w3s_ref[...], preferred_element_type=f32)
    h2 = jnp.maximum(mix(ax, s2, rgx) + b3_ref[...], 0.0)
    hmean = jnp.mean(h2.reshape(_BB, 8, h2.shape[-1]), axis=1)

    pb = pose_ref[...]
    pm = mix(ap, pb, rgp)
    sp1 = jnp.dot(pm.astype(bf16), wp1s_ref[...], preferred_element_type=f32)
    p1 = jnp.maximum(sp1 + bp1_ref[...], 0.0)
    sp2 = jnp.dot(p1.astype(bf16), wp3s_ref[...], preferred_element_type=f32)
    p2 = jnp.maximum(mix(ap, sp2, rgp) + bp3_ref[...], 0.0)
    pmean = jnp.dot(msel_ref[...], p2.astype(bf16), preferred_element_type=f32)

    feat = jnp.concatenate([hmean, pmean], axis=1)
    out_ref[...] = (jnp.dot(feat.astype(bf16), fcws_ref[...],
                            preferred_element_type=f32) + fcb_ref[...])


@jax.jit
def kernel(x, adj, pose, pose_adj, W1, b1, W3, b3, Wp1, bp1, Wp3, bp3, fcW, fcb):
    B = x.shape[0]
    bb = _BB
    nblk = B // bb

    xf = x.reshape(B * 8, x.shape[-1])
    posef = pose.reshape(B * 34, pose.shape[-1])

    bf16 = jnp.bfloat16
    b1r, b3r = b1.reshape(1, -1), b3.reshape(1, -1)
    bp1r, bp3r = bp1.reshape(1, -1), bp3.reshape(1, -1)
    fcbr = fcb.reshape(1, -1)

    msel_np = np.zeros((bb, bb * 34), np.float32)
    for i in range(bb):
        msel_np[i, i * 34:(i + 1) * 34] = 1.0 / 34.0
    msel = jnp.asarray(msel_np, dtype=bf16)

    const2 = lambda i: (0, 0)
    const3 = lambda i: (0, 0, 0)

    out = pl.pallas_call(
        _gcn_kernel,
        grid=(nblk,),
        in_specs=[
            pl.BlockSpec((bb * 8, x.shape[-1]), lambda i: (i, 0)),
            pl.BlockSpec((bb, 8, 8), lambda i: (i, 0, 0)),
            pl.BlockSpec((bb * 34, pose.shape[-1]), lambda i: (i, 0)),
            pl.BlockSpec((bb, 34, 34), lambda i: (i, 0, 0)),
            pl.BlockSpec(W1.shape, const2),
            pl.BlockSpec(b1r.shape, const2),
            pl.BlockSpec(W3.shape, const2),
            pl.BlockSpec(b3r.shape, const2),
            pl.BlockSpec(Wp1.shape, const2),
            pl.BlockSpec(bp1r.shape, const2),
            pl.BlockSpec(Wp3.shape, const2),
            pl.BlockSpec(bp3r.shape, const2),
            pl.BlockSpec(fcW.shape, const2),
            pl.BlockSpec(fcbr.shape, const2),
            pl.BlockSpec(msel.shape, const2),
        ],
        out_specs=pl.BlockSpec((bb, fcW.shape[-1]), lambda i: (i, 0)),
        out_shape=jax.ShapeDtypeStruct((B, fcW.shape[-1]), jnp.float32),
        scratch_shapes=[
            pltpu.VMEM(W1.shape, bf16),
            pltpu.VMEM(W3.shape, bf16),
            pltpu.VMEM(Wp1.shape, bf16),
            pltpu.VMEM(Wp3.shape, bf16),
            pltpu.VMEM(fcW.shape, bf16),
        ],
    )(xf, adj, posef, pose_adj, W1, b1r, W3, b3r, Wp1, bp1r, Wp3, bp3r,
      fcW, fcbr, msel)
    return out


# speedup vs baseline: 3.4673x; 1.1252x over previous
import functools

import jax
import jax.numpy as jnp
import numpy as np
from jax.experimental import pallas as pl
from jax.experimental.pallas import tpu as pltpu

_BB = 32
_GX = 16
_GP = 4


def _blkdiag_bf16(ablk, g, n):
    total = g * n
    acc = None
    for m in range(g):
        pw = ((m * n, total - (m + 1) * n), (m * n, total - (m + 1) * n))
        piece = jnp.pad(ablk[m], pw)
        acc = piece if acc is None else acc + piece
    return acc.astype(jnp.bfloat16)


def _gcn_kernel(x_ref, adj_ref, pose_ref, padj_ref,
                w1_ref, b1_ref, w3_ref, b3_ref,
                wp1_ref, bp1_ref, wp3_ref, bp3_ref,
                fcw_ref, fcb_ref, msel_ref, out_ref,
                w1s_ref, w3s_ref, wp1s_ref, wp3s_ref, fcws_ref):
    f32 = jnp.float32
    bf16 = jnp.bfloat16

    @pl.when(pl.program_id(0) == 0)
    def _cast_weights():
        w1s_ref[...] = w1_ref[...].astype(bf16)
        w3s_ref[...] = w3_ref[...].astype(bf16)
        wp1s_ref[...] = wp1_ref[...].astype(bf16)
        wp3s_ref[...] = wp3_ref[...].astype(bf16)
        fcws_ref[...] = fcw_ref[...].astype(bf16)
    ngx = _BB // _GX
    rgx = _GX * 8
    ngp = _BB // _GP
    rgp = _GP * 34

    def mix(amats, s, rg):
        sb = s.astype(bf16)
        parts = [jnp.dot(a, sb[k * rg:(k + 1) * rg],
                         preferred_element_type=f32)
                 for k, a in enumerate(amats)]
        return jnp.concatenate(parts, axis=0)

    adjb = adj_ref[...]
    ax = [_blkdiag_bf16(adjb[k * _GX:(k + 1) * _GX], _GX, 8)
          for k in range(ngx)]
    padjb = padj_ref[...]
    ap = [_blkdiag_bf16(padjb[k * _GP:(k + 1) * _GP], _GP, 34)
          for k in range(ngp)]

    xb = x_ref[...].astype(bf16)
    s1 = jnp.dot(xb, w1s_ref[...], preferred_element_type=f32)
    h1 = jnp.maximum(mix(ax, s1, rgx) + b1_ref[...], 0.0)
    s2 = jnp.dot(h1.astype(bf16), w3s_ref[...], preferred_element_type=f32)
    h2 = jnp.maximum(mix(ax, s2, rgx) + b3_ref[...], 0.0)
    hmean = jnp.mean(h2.reshape(_BB, 8, h2.shape[-1]), axis=1)

    pb = pose_ref[...].reshape(_BB * 34, pose_ref.shape[-1])
    pm = mix(ap, pb, rgp)
    sp1 = jnp.dot(pm.astype(bf16), wp1s_ref[...], preferred_element_type=f32)
    p1 = jnp.maximum(sp1 + bp1_ref[...], 0.0)
    sp2 = jnp.dot(p1.astype(bf16), wp3s_ref[...], preferred_element_type=f32)
    p2 = jnp.maximum(mix(ap, sp2, rgp) + bp3_ref[...], 0.0)
    pmean = jnp.dot(msel_ref[...], p2.astype(bf16), preferred_element_type=f32)

    feat = jnp.concatenate([hmean, pmean], axis=1)
    out_ref[...] = (jnp.dot(feat.astype(bf16), fcws_ref[...],
                            preferred_element_type=f32) + fcb_ref[...])


@jax.jit
def kernel(x, adj, pose, pose_adj, W1, b1, W3, b3, Wp1, bp1, Wp3, bp3, fcW, fcb):
    B = x.shape[0]
    bb = _BB
    nblk = B // bb

    xf = x.reshape(B * 8, x.shape[-1])

    bf16 = jnp.bfloat16
    b1r, b3r = b1.reshape(1, -1), b3.reshape(1, -1)
    bp1r, bp3r = bp1.reshape(1, -1), bp3.reshape(1, -1)
    fcbr = fcb.reshape(1, -1)

    msel_np = np.zeros((bb, bb * 34), np.float32)
    for i in range(bb):
        msel_np[i, i * 34:(i + 1) * 34] = 1.0 / 34.0
    msel = jnp.asarray(msel_np, dtype=bf16)

    const2 = lambda i: (0, 0)
    const3 = lambda i: (0, 0, 0)

    out = pl.pallas_call(
        _gcn_kernel,
        grid=(nblk,),
        in_specs=[
            pl.BlockSpec((bb * 8, x.shape[-1]), lambda i: (i, 0)),
            pl.BlockSpec((bb, 8, 8), lambda i: (i, 0, 0)),
            pl.BlockSpec((bb, 34, pose.shape[-1]), lambda i: (i, 0, 0)),
            pl.BlockSpec((bb, 34, 34), lambda i: (i, 0, 0)),
            pl.BlockSpec(W1.shape, const2),
            pl.BlockSpec(b1r.shape, const2),
            pl.BlockSpec(W3.shape, const2),
            pl.BlockSpec(b3r.shape, const2),
            pl.BlockSpec(Wp1.shape, const2),
            pl.BlockSpec(bp1r.shape, const2),
            pl.BlockSpec(Wp3.shape, const2),
            pl.BlockSpec(bp3r.shape, const2),
            pl.BlockSpec(fcW.shape, const2),
            pl.BlockSpec(fcbr.shape, const2),
            pl.BlockSpec(msel.shape, const2),
        ],
        out_specs=pl.BlockSpec((bb, fcW.shape[-1]), lambda i: (i, 0)),
        out_shape=jax.ShapeDtypeStruct((B, fcW.shape[-1]), jnp.float32),
        scratch_shapes=[
            pltpu.VMEM(W1.shape, bf16),
            pltpu.VMEM(W3.shape, bf16),
            pltpu.VMEM(Wp1.shape, bf16),
            pltpu.VMEM(Wp3.shape, bf16),
            pltpu.VMEM(fcW.shape, bf16),
        ],
    )(xf, adj, pose, pose_adj, W1, b1r, W3, b3r, Wp1, bp1r, Wp3, bp3r,
      fcW, fcbr, msel)
    return out


# speedup vs baseline: 3.7604x; 1.0845x over previous
import functools

import jax
import jax.numpy as jnp
import numpy as np
from jax.experimental import pallas as pl
from jax.experimental.pallas import tpu as pltpu

_BB = 64
_GX = 16
_GP = 4


def _blkdiag_bf16(ablk, g, n):
    total = g * n
    acc = None
    for m in range(g):
        pw = ((m * n, total - (m + 1) * n), (m * n, total - (m + 1) * n))
        piece = jnp.pad(ablk[m], pw)
        acc = piece if acc is None else acc + piece
    return acc.astype(jnp.bfloat16)


def _gcn_kernel(x_ref, adj_ref, pose_ref, padj_ref,
                w1_ref, b1_ref, w3_ref, b3_ref,
                wp1_ref, bp1_ref, wp3_ref, bp3_ref,
                fcw_ref, fcb_ref, msel_ref, out_ref,
                w1s_ref, w3s_ref, wp1s_ref, wp3s_ref, fcws_ref):
    f32 = jnp.float32
    bf16 = jnp.bfloat16

    @pl.when(pl.program_id(0) == 0)
    def _cast_weights():
        w1s_ref[...] = w1_ref[...].astype(bf16)
        w3s_ref[...] = w3_ref[...].astype(bf16)
        wp1s_ref[...] = wp1_ref[...].astype(bf16)
        wp3s_ref[...] = wp3_ref[...].astype(bf16)
        fcws_ref[...] = fcw_ref[...].astype(bf16)
    ngx = _BB // _GX
    rgx = _GX * 8
    ngp = _BB // _GP
    rgp = _GP * 34

    def mix(amats, s, rg):
        sb = s.astype(bf16)
        parts = [jnp.dot(a, sb[k * rg:(k + 1) * rg],
                         preferred_element_type=f32)
                 for k, a in enumerate(amats)]
        return jnp.concatenate(parts, axis=0)

    adjb = adj_ref[...]
    ax = [_blkdiag_bf16(adjb[k * _GX:(k + 1) * _GX], _GX, 8)
          for k in range(ngx)]
    padjb = padj_ref[...]
    ap = [_blkdiag_bf16(padjb[k * _GP:(k + 1) * _GP], _GP, 34)
          for k in range(ngp)]

    xb = x_ref[...].astype(bf16)
    s1 = jnp.dot(xb, w1s_ref[...], preferred_element_type=f32)
    h1 = jnp.maximum(mix(ax, s1, rgx) + b1_ref[...], 0.0)
    s2 = jnp.dot(h1.astype(bf16), w3s_ref[...], preferred_element_type=f32)
    h2 = jnp.maximum(mix(ax, s2, rgx) + b3_ref[...], 0.0)
    hmean = jnp.mean(h2.reshape(_BB, 8, h2.shape[-1]), axis=1)

    pb = pose_ref[...].reshape(_BB * 34, pose_ref.shape[-1])
    pm = mix(ap, pb, rgp)
    sp1 = jnp.dot(pm.astype(bf16), wp1s_ref[...], preferred_element_type=f32)
    p1 = jnp.maximum(sp1 + bp1_ref[...], 0.0)
    sp2 = jnp.dot(p1.astype(bf16), wp3s_ref[...], preferred_element_type=f32)
    p2 = jnp.maximum(mix(ap, sp2, rgp) + bp3_ref[...], 0.0)
    pmean = jnp.dot(msel_ref[...], p2.astype(bf16), preferred_element_type=f32)

    feat = jnp.concatenate([hmean, pmean], axis=1)
    out_ref[...] = (jnp.dot(feat.astype(bf16), fcws_ref[...],
                            preferred_element_type=f32) + fcb_ref[...])


@jax.jit
def kernel(x, adj, pose, pose_adj, W1, b1, W3, b3, Wp1, bp1, Wp3, bp3, fcW, fcb):
    B = x.shape[0]
    bb = _BB
    nblk = B // bb

    xf = x.reshape(B * 8, x.shape[-1])

    bf16 = jnp.bfloat16
    b1r, b3r = b1.reshape(1, -1), b3.reshape(1, -1)
    bp1r, bp3r = bp1.reshape(1, -1), bp3.reshape(1, -1)
    fcbr = fcb.reshape(1, -1)

    msel_np = np.zeros((bb, bb * 34), np.float32)
    for i in range(bb):
        msel_np[i, i * 34:(i + 1) * 34] = 1.0 / 34.0
    msel = jnp.asarray(msel_np, dtype=bf16)

    const2 = lambda i: (0, 0)
    const3 = lambda i: (0, 0, 0)

    out = pl.pallas_call(
        _gcn_kernel,
        grid=(nblk,),
        in_specs=[
            pl.BlockSpec((bb * 8, x.shape[-1]), lambda i: (i, 0)),
            pl.BlockSpec((bb, 8, 8), lambda i: (i, 0, 0)),
            pl.BlockSpec((bb, 34, pose.shape[-1]), lambda i: (i, 0, 0)),
            pl.BlockSpec((bb, 34, 34), lambda i: (i, 0, 0)),
            pl.BlockSpec(W1.shape, const2),
            pl.BlockSpec(b1r.shape, const2),
            pl.BlockSpec(W3.shape, const2),
            pl.BlockSpec(b3r.shape, const2),
            pl.BlockSpec(Wp1.shape, const2),
            pl.BlockSpec(bp1r.shape, const2),
            pl.BlockSpec(Wp3.shape, const2),
            pl.BlockSpec(bp3r.shape, const2),
            pl.BlockSpec(fcW.shape, const2),
            pl.BlockSpec(fcbr.shape, const2),
            pl.BlockSpec(msel.shape, const2),
        ],
        out_specs=pl.BlockSpec((bb, fcW.shape[-1]), lambda i: (i, 0)),
        out_shape=jax.ShapeDtypeStruct((B, fcW.shape[-1]), jnp.float32),
        scratch_shapes=[
            pltpu.VMEM(W1.shape, bf16),
            pltpu.VMEM(W3.shape, bf16),
            pltpu.VMEM(Wp1.shape, bf16),
            pltpu.VMEM(Wp3.shape, bf16),
            pltpu.VMEM(fcW.shape, bf16),
        ],
    )(xf, adj, pose, pose_adj, W1, b1r, W3, b3r, Wp1, bp1r, Wp3, bp3r,
      fcW, fcbr, msel)
    return out


# speedup vs baseline: 3.8489x; 1.0235x over previous
import functools

import jax
import jax.numpy as jnp
import numpy as np
from jax.experimental import pallas as pl
from jax.experimental.pallas import tpu as pltpu

_BB = 128
_GX = 16
_GP = 4


def _blkdiag_bf16(ablk, g, n):
    total = g * n
    acc = None
    for m in range(g):
        pw = ((m * n, total - (m + 1) * n), (m * n, total - (m + 1) * n))
        piece = jnp.pad(ablk[m], pw)
        acc = piece if acc is None else acc + piece
    return acc.astype(jnp.bfloat16)


def _gcn_kernel(x_ref, adj_ref, pose_ref, padj_ref,
                w1_ref, b1_ref, w3_ref, b3_ref,
                wp1_ref, bp1_ref, wp3_ref, bp3_ref,
                fcw_ref, fcb_ref, msel_ref, out_ref,
                w1s_ref, w3s_ref, wp1s_ref, wp3s_ref, fcws_ref):
    f32 = jnp.float32
    bf16 = jnp.bfloat16

    @pl.when(pl.program_id(0) == 0)
    def _cast_weights():
        w1s_ref[...] = w1_ref[...].astype(bf16)
        w3s_ref[...] = w3_ref[...].astype(bf16)
        wp1s_ref[...] = wp1_ref[...].astype(bf16)
        wp3s_ref[...] = wp3_ref[...].astype(bf16)
        fcws_ref[...] = fcw_ref[...].astype(bf16)
    ngx = _BB // _GX
    rgx = _GX * 8
    ngp = _BB // _GP
    rgp = _GP * 34

    def mix(amats, s, rg):
        sb = s.astype(bf16)
        parts = [jnp.dot(a, sb[k * rg:(k + 1) * rg],
                         preferred_element_type=f32)
                 for k, a in enumerate(amats)]
        return jnp.concatenate(parts, axis=0)

    adjb = adj_ref[...]
    ax = [_blkdiag_bf16(adjb[k * _GX:(k + 1) * _GX], _GX, 8)
          for k in range(ngx)]
    padjb = padj_ref[...]
    ap = [_blkdiag_bf16(padjb[k * _GP:(k + 1) * _GP], _GP, 34)
          for k in range(ngp)]

    xb = x_ref[...].astype(bf16)
    s1 = jnp.dot(xb, w1s_ref[...], preferred_element_type=f32)
    h1 = jnp.maximum(mix(ax, s1, rgx) + b1_ref[...], 0.0)
    s2 = jnp.dot(h1.astype(bf16), w3s_ref[...], preferred_element_type=f32)
    h2 = jnp.maximum(mix(ax, s2, rgx) + b3_ref[...], 0.0)
    hmean = jnp.mean(h2.reshape(_BB, 8, h2.shape[-1]), axis=1)

    pb = pose_ref[...].reshape(_BB * 34, pose_ref.shape[-1])
    pm = mix(ap, pb, rgp)
    sp1 = jnp.dot(pm.astype(bf16), wp1s_ref[...], preferred_element_type=f32)
    p1 = jnp.maximum(sp1 + bp1_ref[...], 0.0)
    sp2 = jnp.dot(p1.astype(bf16), wp3s_ref[...], preferred_element_type=f32)
    p2 = jnp.maximum(mix(ap, sp2, rgp) + bp3_ref[...], 0.0)
    pmean = jnp.dot(msel_ref[...], p2.astype(bf16), preferred_element_type=f32)

    feat = jnp.concatenate([hmean, pmean], axis=1)
    out_ref[...] = (jnp.dot(feat.astype(bf16), fcws_ref[...],
                            preferred_element_type=f32) + fcb_ref[...])


@jax.jit
def kernel(x, adj, pose, pose_adj, W1, b1, W3, b3, Wp1, bp1, Wp3, bp3, fcW, fcb):
    B = x.shape[0]
    bb = _BB
    nblk = B // bb

    xf = x.reshape(B * 8, x.shape[-1])

    bf16 = jnp.bfloat16
    b1r, b3r = b1.reshape(1, -1), b3.reshape(1, -1)
    bp1r, bp3r = bp1.reshape(1, -1), bp3.reshape(1, -1)
    fcbr = fcb.reshape(1, -1)

    msel_np = np.zeros((bb, bb * 34), np.float32)
    for i in range(bb):
        msel_np[i, i * 34:(i + 1) * 34] = 1.0 / 34.0
    msel = jnp.asarray(msel_np, dtype=bf16)

    const2 = lambda i: (0, 0)
    const3 = lambda i: (0, 0, 0)

    out = pl.pallas_call(
        _gcn_kernel,
        grid=(nblk,),
        in_specs=[
            pl.BlockSpec((bb * 8, x.shape[-1]), lambda i: (i, 0)),
            pl.BlockSpec((bb, 8, 8), lambda i: (i, 0, 0)),
            pl.BlockSpec((bb, 34, pose.shape[-1]), lambda i: (i, 0, 0)),
            pl.BlockSpec((bb, 34, 34), lambda i: (i, 0, 0)),
            pl.BlockSpec(W1.shape, const2),
            pl.BlockSpec(b1r.shape, const2),
            pl.BlockSpec(W3.shape, const2),
            pl.BlockSpec(b3r.shape, const2),
            pl.BlockSpec(Wp1.shape, const2),
            pl.BlockSpec(bp1r.shape, const2),
            pl.BlockSpec(Wp3.shape, const2),
            pl.BlockSpec(bp3r.shape, const2),
            pl.BlockSpec(fcW.shape, const2),
            pl.BlockSpec(fcbr.shape, const2),
            pl.BlockSpec(msel.shape, const2),
        ],
        out_specs=pl.BlockSpec((bb, fcW.shape[-1]), lambda i: (i, 0)),
        out_shape=jax.ShapeDtypeStruct((B, fcW.shape[-1]), jnp.float32),
        scratch_shapes=[
            pltpu.VMEM(W1.shape, bf16),
            pltpu.VMEM(W3.shape, bf16),
            pltpu.VMEM(Wp1.shape, bf16),
            pltpu.VMEM(Wp3.shape, bf16),
            pltpu.VMEM(fcW.shape, bf16),
        ],
    )(xf, adj, pose, pose_adj, W1, b1r, W3, b3r, Wp1, bp1r, Wp3, bp3r,
      fcW, fcbr, msel)
    return out


# speedup vs baseline: 5.0701x; 1.3173x over previous
import functools

import jax
import jax.numpy as jnp
import numpy as np
from jax.experimental import pallas as pl
from jax.experimental.pallas import tpu as pltpu

_BB = 128
_GX = 16
_GP = 4


def _blkdiag_bf16(ablk, g, n):
    total = g * n
    acc = None
    for m in range(g):
        pw = ((m * n, total - (m + 1) * n), (m * n, total - (m + 1) * n))
        piece = jnp.pad(ablk[m], pw)
        acc = piece if acc is None else acc + piece
    return acc.astype(jnp.bfloat16)


def _gcn_kernel(x_ref, adj_ref, pose_ref, padj_ref,
                w1_ref, b1_ref, w3_ref, b3_ref,
                wp1_ref, bp1_ref, wp3_ref, bp3_ref,
                fcw_ref, fcb_ref, out_ref,
                w1s_ref, w3s_ref, wp1s_ref, wp3s_ref, msel_ref):
    f32 = jnp.float32
    bf16 = jnp.bfloat16

    @pl.when(pl.program_id(0) == 0)
    def _setup():
        w1s_ref[...] = w1_ref[...].astype(bf16)
        w3s_ref[...] = w3_ref[...].astype(bf16)
        wp1s_ref[...] = wp1_ref[...].astype(bf16)
        wp3s_ref[...] = wp3_ref[...].astype(bf16)
        rows = jax.lax.broadcasted_iota(jnp.int32, (_BB, _BB * 34), 0)
        cols = jax.lax.broadcasted_iota(jnp.int32, (_BB, _BB * 34), 1)
        msel_ref[...] = jnp.where(cols // 34 == rows, 1.0 / 34.0, 0.0
                                  ).astype(bf16)

    ngx = _BB // _GX
    rgx = _GX * 8
    ngp = _BB // _GP
    rgp = _GP * 34

    def mix(amats, s, rg):
        sb = s.astype(bf16)
        parts = [jnp.dot(a, sb[k * rg:(k + 1) * rg],
                         preferred_element_type=f32)
                 for k, a in enumerate(amats)]
        return jnp.concatenate(parts, axis=0)

    adjb = jnp.transpose(adj_ref[...], (2, 0, 1))
    ax = [_blkdiag_bf16(adjb[k * _GX:(k + 1) * _GX], _GX, 8)
          for k in range(ngx)]
    padjb = jnp.transpose(padj_ref[...], (2, 0, 1))
    ap = [_blkdiag_bf16(padjb[k * _GP:(k + 1) * _GP], _GP, 34)
          for k in range(ngp)]

    xb = x_ref[...].astype(bf16)
    s1 = jnp.dot(xb, w1s_ref[...], preferred_element_type=f32)
    h1 = jnp.maximum(mix(ax, s1, rgx) + b1_ref[...], 0.0)
    s2 = jnp.dot(h1.astype(bf16), w3s_ref[...], preferred_element_type=f32)
    h2 = jnp.maximum(mix(ax, s2, rgx) + b3_ref[...], 0.0)
    hmean = jnp.mean(h2.reshape(_BB, 8, h2.shape[-1]), axis=1)

    pb = jnp.transpose(pose_ref[...], (2, 0, 1)).reshape(
        _BB * 34, pose_ref.shape[1])
    pm = mix(ap, pb, rgp)
    sp1 = jnp.dot(pm.astype(bf16), wp1s_ref[...], preferred_element_type=f32)
    p1 = jnp.maximum(sp1 + bp1_ref[...], 0.0)
    sp2 = jnp.dot(p1.astype(bf16), wp3s_ref[...], preferred_element_type=f32)
    p2 = jnp.maximum(mix(ap, sp2, rgp) + bp3_ref[...], 0.0)
    pmean = jnp.dot(msel_ref[...], p2.astype(bf16), preferred_element_type=f32)

    feat = jnp.concatenate([hmean, pmean], axis=1)
    fc_t = jax.lax.dot_general(fcw_ref[...].astype(bf16), feat.astype(bf16),
                               (((1,), (1,)), ((), ())),
                               preferred_element_type=f32)
    out_ref[...] = fc_t + fcb_ref[...]


@jax.jit
def kernel(x, adj, pose, pose_adj, W1, b1, W3, b3, Wp1, bp1, Wp3, bp3, fcW, fcb):
    B = x.shape[0]
    bb = _BB
    nblk = B // bb

    xf = x.reshape(B * 8, x.shape[-1])
    adj_t = jnp.transpose(adj, (1, 2, 0))
    pose_t = jnp.transpose(pose, (1, 2, 0))
    padj_t = jnp.transpose(pose_adj, (1, 2, 0))
    fcw_t = fcW.T

    bf16 = jnp.bfloat16
    b1r, b3r = b1.reshape(1, -1), b3.reshape(1, -1)
    bp1r, bp3r = bp1.reshape(1, -1), bp3.reshape(1, -1)
    fcbr = fcb.reshape(-1, 1)

    const2 = lambda i: (0, 0)

    out = pl.pallas_call(
        _gcn_kernel,
        grid=(nblk,),
        in_specs=[
            pl.BlockSpec((bb * 8, x.shape[-1]), lambda i: (i, 0)),
            pl.BlockSpec((8, 8, bb), lambda i: (0, 0, i)),
            pl.BlockSpec((34, pose.shape[-1], bb), lambda i: (0, 0, i)),
            pl.BlockSpec((34, 34, bb), lambda i: (0, 0, i)),
            pl.BlockSpec(W1.shape, const2),
            pl.BlockSpec(b1r.shape, const2),
            pl.BlockSpec(W3.shape, const2),
            pl.BlockSpec(b3r.shape, const2),
            pl.BlockSpec(Wp1.shape, const2),
            pl.BlockSpec(bp1r.shape, const2),
            pl.BlockSpec(Wp3.shape, const2),
            pl.BlockSpec(bp3r.shape, const2),
            pl.BlockSpec(fcw_t.shape, const2),
            pl.BlockSpec(fcbr.shape, const2),
        ],
        out_specs=pl.BlockSpec((fcW.shape[-1], bb), lambda i: (0, i)),
        out_shape=jax.ShapeDtypeStruct((fcW.shape[-1], B), jnp.float32),
        scratch_shapes=[
            pltpu.VMEM(W1.shape, bf16),
            pltpu.VMEM(W3.shape, bf16),
            pltpu.VMEM(Wp1.shape, bf16),
            pltpu.VMEM(Wp3.shape, bf16),
            pltpu.VMEM((bb, bb * 34), bf16),
        ],
    )(xf, adj_t, pose_t, padj_t, W1, b1r, W3, b3r, Wp1, bp1r, Wp3, bp3r,
      fcw_t, fcbr)
    return out.T
